# baseline (speedup 1.0000x reference)
import functools

import jax
import jax.numpy as jnp
from jax import lax
from jax.experimental import pallas as pl
from jax.experimental.pallas import tpu as pltpu
from jax.experimental.pallas import tpu_sc as plsc

NC = 2
NS = 16
NW = NC * NS
GL = 128
K = 8
CH = K * GL

TBLK = 4096


def _tpose_body(x_ref, o_ref):
    o_ref[...] = x_ref[...].T


@functools.lru_cache(maxsize=None)
def _tpose_wide(rows, cols, blk):
    nblk = pl.cdiv(cols, blk)
    return pl.pallas_call(
        _tpose_body,
        grid=(nblk,),
        in_specs=[pl.BlockSpec((rows, blk), lambda i: (0, i))],
        out_specs=pl.BlockSpec((blk, rows), lambda i: (i, 0)),
        out_shape=jax.ShapeDtypeStruct((cols, rows), jnp.float32),
    )


@functools.lru_cache(maxsize=None)
def _tpose_tall(rows, cols, blk):
    nblk = pl.cdiv(rows, blk)
    return pl.pallas_call(
        _tpose_body,
        grid=(nblk,),
        in_specs=[pl.BlockSpec((blk, cols), lambda i: (i, 0))],
        out_specs=pl.BlockSpec((cols, blk), lambda i: (0, i)),
        out_shape=jax.ShapeDtypeStruct((cols, rows), jnp.float32),
    )


@functools.lru_cache(maxsize=None)
def _gather(v, d, nch):
    mesh = plsc.VectorSubcoreMesh(core_axis_name="c", subcore_axis_name="s")

    @functools.partial(
        pl.kernel,
        mesh=mesh,
        out_type=jax.ShapeDtypeStruct((NW, nch, CH, d), jnp.float32),
        scratch_types=[
            pltpu.VMEM((nch * K, GL), jnp.int32),
            pltpu.VMEM((CH, d), jnp.float32),
            pltpu.SemaphoreType.DMA,
        ],
        compiler_params=pltpu.CompilerParams(use_tc_tiling_on_sc=False),
    )
    def k(table_hbm, tok_hbm, out_hbm, idx_v, rows_v, sem):
        wid = lax.axis_index("s") * NC + lax.axis_index("c")
        pltpu.sync_copy(tok_hbm.at[wid], idx_v)

        def chunk(c, carry):
            cps = [
                pltpu.async_copy(
                    table_hbm.at[idx_v.at[c * K + j]],
                    rows_v.at[pl.ds(j * GL, GL)],
                    sem,
                )
                for j in range(K)
            ]
            for cp in cps:
                cp.wait()
            pltpu.sync_copy(rows_v, out_hbm.at[wid, c])
            return carry

        lax.fori_loop(0, nch, chunk, 0)

    return k


def kernel(tokens, table):
    v, d = table.shape
    flat = tokens.reshape(-1).astype(jnp.int32)
    b = flat.shape[0]
    blk = NW * CH
    pad = (-b) % blk
    if pad:
        flat = jnp.concatenate([flat, jnp.zeros((pad,), jnp.int32)])
    nch = flat.shape[0] // blk
    tok3 = flat.reshape(NW, nch * K, GL)

    table_rm = _tpose_wide(d, v, TBLK)(table.T)
    rows = _gather(v, d, nch)(table_rm, tok3)
    rows2 = rows.reshape(-1, d)
    out_t = _tpose_tall(rows2.shape[0], d, TBLK)(rows2)
    out = out_t.T
    if pad:
        out = out[:b]
    return out
